# baseline (speedup 1.0000x reference)
import functools
import jax
import jax.numpy as jnp
from jax import lax
from jax.experimental import pallas as pl
from jax.experimental.pallas import tpu as pltpu
from jax.experimental.pallas import tpu_sc as plsc

N = 10000
NP = 10240
E = 320000
G = 64
NW = 32
TE = E // NW
EB = 80
TEP = 10080
NEB = TEP // EB
RPT = NP // 16
PPT = 128 // 16
F = 128
R = 512
NEG = 0.03

_mesh = plsc.VectorSubcoreMesh(core_axis_name="c", subcore_axis_name="s")


@functools.partial(
    pl.kernel,
    out_type=(jax.ShapeDtypeStruct((2, NP), jnp.float32),
              jax.ShapeDtypeStruct((2, 128), jnp.float32)),
    mesh=_mesh,
    scratch_types=[
        pltpu.VMEM((NEB, EB), jnp.int32),
        pltpu.VMEM((4, 80), jnp.int32),
        pltpu.VMEM((128,), jnp.float32),
        pltpu.VMEM_SHARED((NP,), jnp.float32),
        pltpu.VMEM_SHARED((128,), jnp.float32),
    ],
)
def _sc_deg(dsts_hbm, batch_hbm, ones_hbm, z1_hbm, deg_hbm, cnt_hbm,
            idx_d, bidx, ones_v, sdeg, scnt):
    c = lax.axis_index("c")
    s = lax.axis_index("s")
    wid = c * 16 + s
    pltpu.sync_copy(z1_hbm.at[pl.ds(0, RPT)], sdeg.at[pl.ds(s * RPT, RPT)])

    @pl.when(s == 0)
    def _():
        pltpu.sync_copy(z1_hbm.at[pl.ds(0, 128)], scnt)

    pltpu.sync_copy(dsts_hbm.at[wid], idx_d)
    pltpu.sync_copy(batch_hbm.at[wid], bidx)
    pltpu.sync_copy(ones_hbm, ones_v)
    plsc.subcore_barrier()

    def deg_body(jb, carry):
        pltpu.sync_copy(ones_v.at[pl.ds(0, EB)], sdeg.at[idx_d.at[jb]], add=True)
        return carry

    lax.fori_loop(0, NEB, deg_body, 0, unroll=False)

    def cnt_body(jb, carry):
        pltpu.sync_copy(ones_v.at[pl.ds(0, 80)], scnt.at[bidx.at[jb]], add=True)
        return carry

    lax.fori_loop(0, 4, cnt_body, 0, unroll=False)
    plsc.subcore_barrier()
    pltpu.sync_copy(sdeg.at[pl.ds(s * RPT, RPT)], deg_hbm.at[c, pl.ds(s * RPT, RPT)])

    @pl.when(s == 0)
    def _():
        pltpu.sync_copy(scnt, cnt_hbm.at[c])


def _make_mp(nch):
    @functools.partial(
        pl.kernel,
        out_type=tuple(jax.ShapeDtypeStruct((2, NP, F), jnp.float32)
                       for _ in range(nch)),
        mesh=_mesh,
        compiler_params=pltpu.CompilerParams(use_tc_tiling_on_sc=False),
        scratch_types=[
            pltpu.VMEM((NEB, EB), jnp.int32),
            pltpu.VMEM((NEB, EB), jnp.int32),
            pltpu.VMEM((EB, F), jnp.float32),
            pltpu.VMEM((EB, F), jnp.float32),
            pltpu.VMEM_SHARED((NP, F), jnp.float32),
            pltpu.SemaphoreType.DMA,
            pltpu.SemaphoreType.DMA,
        ],
    )
    def mp(*refs):
        h_hbms = refs[:nch]
        srcs_hbm, dsts_hbm, z2_hbm = refs[nch:nch + 3]
        p_hbms = refs[nch + 3:2 * nch + 3]
        idx_s, idx_d, rows0, rows1, acc, gs0, gs1 = refs[2 * nch + 3:]
        c = lax.axis_index("c")
        s = lax.axis_index("s")
        wid = c * 16 + s
        pltpu.sync_copy(srcs_hbm.at[wid], idx_s)
        pltpu.sync_copy(dsts_hbm.at[wid], idx_d)
        for k in range(nch):
            h_hbm = h_hbms[k]
            pltpu.sync_copy(z2_hbm, acc.at[pl.ds(s * RPT, RPT)])
            pltpu.async_copy(h_hbm.at[idx_s.at[0]], rows0, gs0)
            plsc.subcore_barrier()

            def body(j2, carry):
                jb0 = j2 * 2
                jb1 = jb0 + 1
                pltpu.async_copy(h_hbm.at[idx_s.at[jb1]], rows1, gs1)
                pltpu.make_async_copy(h_hbm.at[idx_s.at[jb0]], rows0, gs0).wait()
                pltpu.sync_copy(rows0, acc.at[idx_d.at[jb0]], add=True)

                @pl.when(jb1 + 1 < NEB)
                def _():
                    pltpu.async_copy(h_hbm.at[idx_s.at[jb1 + 1]], rows0, gs0)

                pltpu.make_async_copy(h_hbm.at[idx_s.at[jb1]], rows1, gs1).wait()
                pltpu.sync_copy(rows1, acc.at[idx_d.at[jb1]], add=True)
                return carry

            lax.fori_loop(0, NEB // 2, body, 0, unroll=False)
            plsc.subcore_barrier()
            pltpu.sync_copy(acc.at[pl.ds(s * RPT, RPT)],
                            p_hbms[k].at[c, pl.ds(s * RPT, RPT)])

    return mp


_mp2 = _make_mp(2)
_mp1 = _make_mp(1)


@functools.partial(
    pl.kernel,
    out_type=jax.ShapeDtypeStruct((2, 128, 128), jnp.float32),
    mesh=_mesh,
    scratch_types=[
        pltpu.VMEM((4, 80), jnp.int32),
        pltpu.VMEM((80, 128), jnp.float32),
        pltpu.VMEM_SHARED((128, 128), jnp.float32),
    ],
)
def _sc_pool(h_hbm, batch_hbm, z2_hbm, pp_hbm, bidx, rows, acc):
    c = lax.axis_index("c")
    s = lax.axis_index("s")
    wid = c * 16 + s
    pltpu.sync_copy(z2_hbm.at[pl.ds(0, PPT)], acc.at[pl.ds(s * PPT, PPT)])
    pltpu.sync_copy(batch_hbm.at[wid], bidx)
    plsc.subcore_barrier()

    def body(jb, carry):
        pltpu.sync_copy(h_hbm.at[pl.ds(wid * 320 + jb * 80, 80)], rows)
        pltpu.sync_copy(rows, acc.at[bidx.at[jb]], add=True)
        return carry

    lax.fori_loop(0, 4, body, 0, unroll=False)
    plsc.subcore_barrier()
    pltpu.sync_copy(acc.at[pl.ds(s * PPT, PPT)],
                    pp_hbm.at[c, pl.ds(s * PPT, PPT)])


def _dinv_blk(deg_ref):
    d = deg_ref[0] + deg_ref[1] + 1.0
    return lax.rsqrt(d)


def _leaky(a):
    return jnp.where(a >= 0, a, NEG * a)


def _k_pre0(x_ref, w_ref, deg_ref, *o_refs):
    dinv = _dinv_blk(deg_ref)
    hs = jnp.dot(x_ref[...], w_ref[...], preferred_element_type=jnp.float32) * dinv
    for k, o in enumerate(o_refs):
        o[...] = hs[:, k * F:(k + 1) * F]


def _k_stats(pa_ref, pb_ref, ha_ref, hb_ref, deg_ref, b_ref,
             conv_ref, sum_ref, sq_ref):
    i = pl.program_id(0)
    dinv = _dinv_blk(deg_ref)
    ca = (pa_ref[0] + pa_ref[1] + ha_ref[...]) * dinv
    cb = (pb_ref[0] + pb_ref[1] + hb_ref[...]) * dinv
    conv = jnp.concatenate([ca, cb], axis=1) + b_ref[...]
    conv_ref[...] = conv
    rows = lax.broadcasted_iota(jnp.int32, (R, 1), 0) + i * R
    mask = (rows < N).astype(jnp.float32)
    cm = conv * mask

    @pl.when(i == 0)
    def _():
        sum_ref[...] = jnp.zeros_like(sum_ref)
        sq_ref[...] = jnp.zeros_like(sq_ref)

    sum_ref[...] += jnp.sum(cm, axis=0, keepdims=True)
    sq_ref[...] += jnp.sum(conv * cm, axis=0, keepdims=True)


def _bn_leaky(conv_ref, sum_ref, sq_ref, g_ref, bt_ref):
    m = sum_ref[...] / N
    v = sq_ref[...] / N - m * m
    a = (conv_ref[...] - m) * lax.rsqrt(v + 1e-5) * g_ref[...] + bt_ref[...]
    return _leaky(a)


def _k_pre_mid(conv_ref, sum_ref, sq_ref, g_ref, bt_ref, w_ref, deg_ref,
               *o_refs):
    dinv = _dinv_blk(deg_ref)
    a = _bn_leaky(conv_ref, sum_ref, sq_ref, g_ref, bt_ref)
    hs = jnp.dot(a, w_ref[...], preferred_element_type=jnp.float32) * dinv
    for k, o in enumerate(o_refs):
        o[...] = hs[:, k * F:(k + 1) * F]


def _k_post2(p_ref, h_ref, deg_ref, b_ref, o_ref):
    dinv = _dinv_blk(deg_ref)
    conv = (p_ref[0] + p_ref[1] + h_ref[...]) * dinv + b_ref[...]
    o_ref[...] = _leaky(conv)


def _k_head(pp_ref, cnt_ref, lw0_ref, lb0_ref, lw1_ref, lb1_ref, o_ref):
    spool = pp_ref[0, :G, :] + pp_ref[1, :G, :]
    cc = cnt_ref[0, :G, :] + cnt_ref[1, :G, :]
    pool = spool / jnp.maximum(cc, 1.0)
    h = jnp.dot(pool, lw0_ref[...], preferred_element_type=jnp.float32) + lb0_ref[...]
    o_ref[...] = jnp.dot(h, lw1_ref[...], preferred_element_type=jnp.float32) + lb1_ref[...]


def _rows_spec(cols):
    return pl.BlockSpec((R, cols), lambda i: (i, 0))


def _full_spec(shape):
    return pl.BlockSpec(shape, lambda i: tuple(0 for _ in shape))


_deg_spec = pl.BlockSpec((2, R, 1), lambda i: (0, i, 0))
_p_spec = pl.BlockSpec((2, R, F), lambda i: (0, i, 0))
_grid = (NP // R,)


def kernel(x, edge_index, batch, cw0, cb0, cw1, cb1, cw2, cb2,
           bg0, bb0, bg1, bb1, lw0, lb0, lw1, lb1):
    f32 = jnp.float32
    i32 = jnp.int32
    x_pad = jnp.concatenate([x, jnp.zeros((NP - N, x.shape[1]), f32)], axis=0)
    srcs = jnp.concatenate(
        [edge_index[0].reshape(NW, TE), jnp.zeros((NW, TEP - TE), i32)],
        axis=1).reshape(NW, NEB, EB)
    dsts = jnp.concatenate(
        [edge_index[1].reshape(NW, TE), jnp.full((NW, TEP - TE), NP - 1, i32)],
        axis=1).reshape(NW, NEB, EB)
    batch_r = jnp.concatenate(
        [batch, jnp.full((NP - N,), G, i32)]).reshape(NW, 4, 80)
    ones128 = jnp.ones((128,), f32)
    z1 = jnp.zeros((RPT,), f32)
    z2 = jnp.zeros((RPT, F), f32)

    deg, cnt = _sc_deg(dsts, batch_r, ones128, z1)
    degv = deg.reshape(2, NP, 1)
    cntv = cnt.reshape(2, 128, 1)

    stats_specs = dict(
        in_specs=[_p_spec, _p_spec, _rows_spec(F), _rows_spec(F), _deg_spec,
                  _full_spec((1, 256))],
        out_specs=[_rows_spec(256), _full_spec((1, 256)), _full_spec((1, 256))],
        out_shape=[jax.ShapeDtypeStruct((NP, 256), f32),
                   jax.ShapeDtypeStruct((1, 256), f32),
                   jax.ShapeDtypeStruct((1, 256), f32)],
    )

    h0 = pl.pallas_call(
        _k_pre0,
        grid=_grid,
        in_specs=[_rows_spec(128), _full_spec((128, 256)), _deg_spec],
        out_specs=[_rows_spec(F)] * 2,
        out_shape=[jax.ShapeDtypeStruct((NP, F), f32)] * 2,
    )(x_pad, cw0, degv)
    p0 = _mp2(*h0, srcs, dsts, z2)
    conv0, sum0, sq0 = pl.pallas_call(
        _k_stats, grid=_grid, **stats_specs,
    )(*p0, *h0, degv, cb0.reshape(1, 256))

    h1 = pl.pallas_call(
        _k_pre_mid,
        grid=_grid,
        in_specs=[_rows_spec(256), _full_spec((1, 256)), _full_spec((1, 256)),
                  _full_spec((1, 256)), _full_spec((1, 256)),
                  _full_spec((256, 256)), _deg_spec],
        out_specs=[_rows_spec(F)] * 2,
        out_shape=[jax.ShapeDtypeStruct((NP, F), f32)] * 2,
    )(conv0, sum0, sq0, bg0.reshape(1, 256), bb0.reshape(1, 256), cw1, degv)
    p1 = _mp2(*h1, srcs, dsts, z2)
    conv1, sum1, sq1 = pl.pallas_call(
        _k_stats, grid=_grid, **stats_specs,
    )(*p1, *h1, degv, cb1.reshape(1, 256))

    h2s = pl.pallas_call(
        _k_pre_mid,
        grid=_grid,
        in_specs=[_rows_spec(256), _full_spec((1, 256)), _full_spec((1, 256)),
                  _full_spec((1, 256)), _full_spec((1, 256)),
                  _full_spec((256, 128)), _deg_spec],
        out_specs=[_rows_spec(F)],
        out_shape=[jax.ShapeDtypeStruct((NP, F), f32)],
    )(conv1, sum1, sq1, bg1.reshape(1, 256), bb1.reshape(1, 256), cw2, degv)
    p2 = _mp1(*h2s, srcs, dsts, z2)
    h2 = pl.pallas_call(
        _k_post2,
        grid=_grid,
        in_specs=[_p_spec, _rows_spec(F), _deg_spec, _full_spec((1, 128))],
        out_specs=_rows_spec(128),
        out_shape=jax.ShapeDtypeStruct((NP, 128), f32),
    )(p2[0], h2s[0], degv, cb2.reshape(1, 128))

    pp = _sc_pool(h2, batch_r, z2)
    out = pl.pallas_call(
        _k_head,
        in_specs=[pl.BlockSpec((2, 128, 128), lambda: (0, 0, 0)),
                  pl.BlockSpec((2, 128, 1), lambda: (0, 0, 0)),
                  pl.BlockSpec((128, 64), lambda: (0, 0)),
                  pl.BlockSpec((1, 64), lambda: (0, 0)),
                  pl.BlockSpec((64, 16), lambda: (0, 0)),
                  pl.BlockSpec((1, 16), lambda: (0, 0))],
        out_specs=pl.BlockSpec((G, 16), lambda: (0, 0)),
        out_shape=jax.ShapeDtypeStruct((G, 16), f32),
    )(pp, cntv, lw0, lb0.reshape(1, 64), lw1, lb1.reshape(1, 16))
    return out


# speedup vs baseline: 1.4714x; 1.4714x over previous
import functools
import jax
import jax.numpy as jnp
from jax import lax
from jax.experimental import pallas as pl
from jax.experimental.pallas import tpu as pltpu
from jax.experimental.pallas import tpu_sc as plsc

N = 10000
NP = 10240
E = 320000
G = 64
NW = 32
TE = E // NW
EB = 125
NEB = TE // EB
RPT = NP // 16
PPT = 128 // 16
F = 64
R = 512
NEG = 0.03

_mesh = plsc.VectorSubcoreMesh(core_axis_name="c", subcore_axis_name="s")


@functools.partial(
    pl.kernel,
    out_type=(jax.ShapeDtypeStruct((2, NP), jnp.float32),
              jax.ShapeDtypeStruct((2, 128), jnp.float32)),
    mesh=_mesh,
    scratch_types=[
        pltpu.VMEM((NEB, EB), jnp.int32),
        pltpu.VMEM((4, 80), jnp.int32),
        pltpu.VMEM((128,), jnp.float32),
        pltpu.VMEM_SHARED((NP,), jnp.float32),
        pltpu.VMEM_SHARED((128,), jnp.float32),
    ],
)
def _sc_deg(dsts_hbm, batch_hbm, ones_hbm, z1_hbm, deg_hbm, cnt_hbm,
            idx_d, bidx, ones_v, sdeg, scnt):
    c = lax.axis_index("c")
    s = lax.axis_index("s")
    wid = c * 16 + s
    pltpu.sync_copy(z1_hbm.at[pl.ds(0, RPT)], sdeg.at[pl.ds(s * RPT, RPT)])

    @pl.when(s == 0)
    def _():
        pltpu.sync_copy(z1_hbm.at[pl.ds(0, 128)], scnt)

    pltpu.sync_copy(dsts_hbm.at[wid], idx_d)
    pltpu.sync_copy(batch_hbm.at[wid], bidx)
    pltpu.sync_copy(ones_hbm, ones_v)
    plsc.subcore_barrier()

    def deg_body(jb, carry):
        pltpu.sync_copy(ones_v.at[pl.ds(0, EB)], sdeg.at[idx_d.at[jb]], add=True)
        return carry

    lax.fori_loop(0, NEB, deg_body, 0, unroll=False)

    def cnt_body(jb, carry):
        pltpu.sync_copy(ones_v.at[pl.ds(0, 80)], scnt.at[bidx.at[jb]], add=True)
        return carry

    lax.fori_loop(0, 4, cnt_body, 0, unroll=False)
    plsc.subcore_barrier()
    pltpu.sync_copy(sdeg.at[pl.ds(s * RPT, RPT)], deg_hbm.at[c, pl.ds(s * RPT, RPT)])

    @pl.when(s == 0)
    def _():
        pltpu.sync_copy(scnt, cnt_hbm.at[c])


def _make_mp(nch):
    @functools.partial(
        pl.kernel,
        out_type=tuple(jax.ShapeDtypeStruct((2, NP, F), jnp.float32)
                       for _ in range(nch)),
        mesh=_mesh,
        compiler_params=pltpu.CompilerParams(use_tc_tiling_on_sc=False),
        scratch_types=[
            pltpu.VMEM((NEB, EB), jnp.int32),
            pltpu.VMEM((NEB, EB), jnp.int32),
            pltpu.VMEM((EB, F), jnp.float32),
            pltpu.VMEM((EB, F), jnp.float32),
            pltpu.VMEM((EB, F), jnp.float32),
            pltpu.VMEM((EB, F), jnp.float32),
            pltpu.VMEM_SHARED((NP, F), jnp.float32),
            pltpu.SemaphoreType.DMA,
            pltpu.SemaphoreType.DMA,
            pltpu.SemaphoreType.DMA,
            pltpu.SemaphoreType.DMA,
        ],
    )
    def mp(*refs):
        h_hbms = refs[:nch]
        srcs_hbm, dsts_hbm, z2_hbm = refs[nch:nch + 3]
        p_hbms = refs[nch + 3:2 * nch + 3]
        idx_s, idx_d = refs[2 * nch + 3:2 * nch + 5]
        rows = refs[2 * nch + 5:2 * nch + 9]
        acc = refs[2 * nch + 9]
        gs = refs[2 * nch + 10:2 * nch + 14]
        c = lax.axis_index("c")
        s = lax.axis_index("s")
        wid = c * 16 + s
        pltpu.sync_copy(srcs_hbm.at[wid], idx_s)
        pltpu.sync_copy(dsts_hbm.at[wid], idx_d)
        for k in range(nch):
            h_hbm = h_hbms[k]
            pltpu.sync_copy(z2_hbm, acc.at[pl.ds(s * RPT, RPT)])
            for t in range(3):
                pltpu.async_copy(h_hbm.at[idx_s.at[t]], rows[t], gs[t])
            plsc.subcore_barrier()

            def body(j4, carry):
                base = j4 * 4
                for t in range(4):
                    jb = base + t
                    pltpu.make_async_copy(
                        h_hbm.at[idx_s.at[jb]], rows[t], gs[t]).wait()
                    pltpu.sync_copy(rows[t], acc.at[idx_d.at[jb]], add=True)
                    tn = (t + 3) % 4

                    @pl.when(jb + 3 < NEB)
                    def _():
                        pltpu.async_copy(
                            h_hbm.at[idx_s.at[jb + 3]], rows[tn], gs[tn])

                return carry

            lax.fori_loop(0, NEB // 4, body, 0, unroll=False)
            plsc.subcore_barrier()
            pltpu.sync_copy(acc.at[pl.ds(s * RPT, RPT)],
                            p_hbms[k].at[c, pl.ds(s * RPT, RPT)])

    return mp


_mp4 = _make_mp(4)
_mp2 = _make_mp(2)


@functools.partial(
    pl.kernel,
    out_type=jax.ShapeDtypeStruct((2, 128, 128), jnp.float32),
    mesh=_mesh,
    scratch_types=[
        pltpu.VMEM((4, 80), jnp.int32),
        pltpu.VMEM((80, 128), jnp.float32),
        pltpu.VMEM_SHARED((128, 128), jnp.float32),
    ],
)
def _sc_pool(h_hbm, batch_hbm, z2_hbm, pp_hbm, bidx, rows, acc):
    c = lax.axis_index("c")
    s = lax.axis_index("s")
    wid = c * 16 + s
    pltpu.sync_copy(z2_hbm.at[pl.ds(0, PPT)], acc.at[pl.ds(s * PPT, PPT)])
    pltpu.sync_copy(batch_hbm.at[wid], bidx)
    plsc.subcore_barrier()

    def body(jb, carry):
        pltpu.sync_copy(h_hbm.at[pl.ds(wid * 320 + jb * 80, 80)], rows)
        pltpu.sync_copy(rows, acc.at[bidx.at[jb]], add=True)
        return carry

    lax.fori_loop(0, 4, body, 0, unroll=False)
    plsc.subcore_barrier()
    pltpu.sync_copy(acc.at[pl.ds(s * PPT, PPT)],
                    pp_hbm.at[c, pl.ds(s * PPT, PPT)])


def _dinv_blk(deg_ref):
    d = deg_ref[0] + deg_ref[1] + 1.0
    return lax.rsqrt(d)


def _leaky(a):
    return jnp.where(a >= 0, a, NEG * a)


def _k_pre0(x_ref, w_ref, deg_ref, *o_refs):
    dinv = _dinv_blk(deg_ref)
    hs = jnp.dot(x_ref[...], w_ref[...], preferred_element_type=jnp.float32) * dinv
    for k, o in enumerate(o_refs):
        o[...] = hs[:, k * F:(k + 1) * F]


def _k_stats(*refs):
    p_refs = refs[:4]
    h_refs = refs[4:8]
    deg_ref, b_ref, conv_ref, sum_ref, sq_ref = refs[8:]
    i = pl.program_id(0)
    dinv = _dinv_blk(deg_ref)
    parts = [(p[0] + p[1] + h[...]) * dinv for p, h in zip(p_refs, h_refs)]
    conv = jnp.concatenate(parts, axis=1) + b_ref[...]
    conv_ref[...] = conv
    rows = lax.broadcasted_iota(jnp.int32, (R, 1), 0) + i * R
    mask = (rows < N).astype(jnp.float32)
    cm = conv * mask

    @pl.when(i == 0)
    def _():
        sum_ref[...] = jnp.zeros_like(sum_ref)
        sq_ref[...] = jnp.zeros_like(sq_ref)

    sum_ref[...] += jnp.sum(cm, axis=0, keepdims=True)
    sq_ref[...] += jnp.sum(conv * cm, axis=0, keepdims=True)


def _bn_leaky(conv_ref, sum_ref, sq_ref, g_ref, bt_ref):
    m = sum_ref[...] / N
    v = sq_ref[...] / N - m * m
    a = (conv_ref[...] - m) * lax.rsqrt(v + 1e-5) * g_ref[...] + bt_ref[...]
    return _leaky(a)


def _k_pre_mid(conv_ref, sum_ref, sq_ref, g_ref, bt_ref, w_ref, deg_ref,
               *o_refs):
    dinv = _dinv_blk(deg_ref)
    a = _bn_leaky(conv_ref, sum_ref, sq_ref, g_ref, bt_ref)
    hs = jnp.dot(a, w_ref[...], preferred_element_type=jnp.float32) * dinv
    for k, o in enumerate(o_refs):
        o[...] = hs[:, k * F:(k + 1) * F]


def _k_post2(pa_ref, pb_ref, ha_ref, hb_ref, deg_ref, b_ref, o_ref):
    dinv = _dinv_blk(deg_ref)
    ca = (pa_ref[0] + pa_ref[1] + ha_ref[...]) * dinv
    cb = (pb_ref[0] + pb_ref[1] + hb_ref[...]) * dinv
    conv = jnp.concatenate([ca, cb], axis=1) + b_ref[...]
    o_ref[...] = _leaky(conv)


def _k_head(pp_ref, cnt_ref, lw0_ref, lb0_ref, lw1_ref, lb1_ref, o_ref):
    spool = pp_ref[0, :G, :] + pp_ref[1, :G, :]
    cc = cnt_ref[0, :G, :] + cnt_ref[1, :G, :]
    pool = spool / jnp.maximum(cc, 1.0)
    h = jnp.dot(pool, lw0_ref[...], preferred_element_type=jnp.float32) + lb0_ref[...]
    o_ref[...] = jnp.dot(h, lw1_ref[...], preferred_element_type=jnp.float32) + lb1_ref[...]


def _rows_spec(cols):
    return pl.BlockSpec((R, cols), lambda i: (i, 0))


def _full_spec(shape):
    return pl.BlockSpec(shape, lambda i: tuple(0 for _ in shape))


_deg_spec = pl.BlockSpec((2, R, 1), lambda i: (0, i, 0))
_p_spec = pl.BlockSpec((2, R, F), lambda i: (0, i, 0))
_grid = (NP // R,)


def kernel(x, edge_index, batch, cw0, cb0, cw1, cb1, cw2, cb2,
           bg0, bb0, bg1, bb1, lw0, lb0, lw1, lb1):
    f32 = jnp.float32
    x_pad = jnp.concatenate([x, jnp.zeros((NP - N, x.shape[1]), f32)], axis=0)
    srcs = edge_index[0].reshape(NW, NEB, EB)
    dsts = edge_index[1].reshape(NW, NEB, EB)
    batch_r = jnp.concatenate(
        [batch, jnp.full((NP - N,), G, jnp.int32)]).reshape(NW, 4, 80)
    ones128 = jnp.ones((128,), f32)
    z1 = jnp.zeros((RPT,), f32)
    z2 = jnp.zeros((RPT, F), f32)
    z2w = jnp.zeros((RPT, 128), f32)

    deg, cnt = _sc_deg(dsts, batch_r, ones128, z1)
    degv = deg.reshape(2, NP, 1)
    cntv = cnt.reshape(2, 128, 1)

    stats_specs = dict(
        in_specs=[_p_spec] * 4 + [_rows_spec(F)] * 4 + [_deg_spec,
                  _full_spec((1, 256))],
        out_specs=[_rows_spec(256), _full_spec((1, 256)), _full_spec((1, 256))],
        out_shape=[jax.ShapeDtypeStruct((NP, 256), f32),
                   jax.ShapeDtypeStruct((1, 256), f32),
                   jax.ShapeDtypeStruct((1, 256), f32)],
    )

    h0 = pl.pallas_call(
        _k_pre0,
        grid=_grid,
        in_specs=[_rows_spec(128), _full_spec((128, 256)), _deg_spec],
        out_specs=[_rows_spec(F)] * 4,
        out_shape=[jax.ShapeDtypeStruct((NP, F), f32)] * 4,
    )(x_pad, cw0, degv)
    p0 = _mp4(*h0, srcs, dsts, z2)
    conv0, sum0, sq0 = pl.pallas_call(
        _k_stats, grid=_grid, **stats_specs,
    )(*p0, *h0, degv, cb0.reshape(1, 256))

    h1 = pl.pallas_call(
        _k_pre_mid,
        grid=_grid,
        in_specs=[_rows_spec(256), _full_spec((1, 256)), _full_spec((1, 256)),
                  _full_spec((1, 256)), _full_spec((1, 256)),
                  _full_spec((256, 256)), _deg_spec],
        out_specs=[_rows_spec(F)] * 4,
        out_shape=[jax.ShapeDtypeStruct((NP, F), f32)] * 4,
    )(conv0, sum0, sq0, bg0.reshape(1, 256), bb0.reshape(1, 256), cw1, degv)
    p1 = _mp4(*h1, srcs, dsts, z2)
    conv1, sum1, sq1 = pl.pallas_call(
        _k_stats, grid=_grid, **stats_specs,
    )(*p1, *h1, degv, cb1.reshape(1, 256))

    h2s = pl.pallas_call(
        _k_pre_mid,
        grid=_grid,
        in_specs=[_rows_spec(256), _full_spec((1, 256)), _full_spec((1, 256)),
                  _full_spec((1, 256)), _full_spec((1, 256)),
                  _full_spec((256, 128)), _deg_spec],
        out_specs=[_rows_spec(F)] * 2,
        out_shape=[jax.ShapeDtypeStruct((NP, F), f32)] * 2,
    )(conv1, sum1, sq1, bg1.reshape(1, 256), bb1.reshape(1, 256), cw2, degv)
    p2 = _mp2(*h2s, srcs, dsts, z2)
    h2 = pl.pallas_call(
        _k_post2,
        grid=_grid,
        in_specs=[_p_spec, _p_spec, _rows_spec(F), _rows_spec(F), _deg_spec,
                  _full_spec((1, 128))],
        out_specs=_rows_spec(128),
        out_shape=jax.ShapeDtypeStruct((NP, 128), f32),
    )(*p2, *h2s, degv, cb2.reshape(1, 128))

    pp = _sc_pool(h2, batch_r, z2w)
    out = pl.pallas_call(
        _k_head,
        in_specs=[pl.BlockSpec((2, 128, 128), lambda: (0, 0, 0)),
                  pl.BlockSpec((2, 128, 1), lambda: (0, 0, 0)),
                  pl.BlockSpec((128, 64), lambda: (0, 0)),
                  pl.BlockSpec((1, 64), lambda: (0, 0)),
                  pl.BlockSpec((64, 16), lambda: (0, 0)),
                  pl.BlockSpec((1, 16), lambda: (0, 0))],
        out_specs=pl.BlockSpec((G, 16), lambda: (0, 0)),
        out_shape=jax.ShapeDtypeStruct((G, 16), f32),
    )(pp, cntv, lw0, lb0.reshape(1, 64), lw1, lb1.reshape(1, 16))
    return out


# speedup vs baseline: 1.4896x; 1.0123x over previous
import functools
import jax
import jax.numpy as jnp
from jax import lax
from jax.experimental import pallas as pl
from jax.experimental.pallas import tpu as pltpu
from jax.experimental.pallas import tpu_sc as plsc

N = 10000
NP = 10240
E = 320000
G = 64
NW = 32
TE = E // NW
EB = 125
NEB = TE // EB
RPT = NP // 16
PPT = 128 // 16
F = 64
NBUF = 8
R = 512
NEG = 0.03

_mesh = plsc.VectorSubcoreMesh(core_axis_name="c", subcore_axis_name="s")


@functools.partial(
    pl.kernel,
    out_type=(jax.ShapeDtypeStruct((2, NP), jnp.float32),
              jax.ShapeDtypeStruct((2, 128), jnp.float32)),
    mesh=_mesh,
    scratch_types=[
        pltpu.VMEM((NEB, EB), jnp.int32),
        pltpu.VMEM((4, 80), jnp.int32),
        pltpu.VMEM((128,), jnp.float32),
        pltpu.VMEM_SHARED((NP,), jnp.float32),
        pltpu.VMEM_SHARED((128,), jnp.float32),
    ],
)
def _sc_deg(dsts_hbm, batch_hbm, ones_hbm, z1_hbm, deg_hbm, cnt_hbm,
            idx_d, bidx, ones_v, sdeg, scnt):
    c = lax.axis_index("c")
    s = lax.axis_index("s")
    wid = c * 16 + s
    pltpu.sync_copy(z1_hbm.at[pl.ds(0, RPT)], sdeg.at[pl.ds(s * RPT, RPT)])

    @pl.when(s == 0)
    def _():
        pltpu.sync_copy(z1_hbm.at[pl.ds(0, 128)], scnt)

    pltpu.sync_copy(dsts_hbm.at[wid], idx_d)
    pltpu.sync_copy(batch_hbm.at[wid], bidx)
    pltpu.sync_copy(ones_hbm, ones_v)
    plsc.subcore_barrier()

    def deg_body(jb, carry):
        pltpu.sync_copy(ones_v.at[pl.ds(0, EB)], sdeg.at[idx_d.at[jb]], add=True)
        return carry

    lax.fori_loop(0, NEB, deg_body, 0, unroll=False)

    def cnt_body(jb, carry):
        pltpu.sync_copy(ones_v.at[pl.ds(0, 80)], scnt.at[bidx.at[jb]], add=True)
        return carry

    lax.fori_loop(0, 4, cnt_body, 0, unroll=False)
    plsc.subcore_barrier()
    pltpu.sync_copy(sdeg.at[pl.ds(s * RPT, RPT)], deg_hbm.at[c, pl.ds(s * RPT, RPT)])

    @pl.when(s == 0)
    def _():
        pltpu.sync_copy(scnt, cnt_hbm.at[c])


def _make_mp(nch):
    @functools.partial(
        pl.kernel,
        out_type=tuple(jax.ShapeDtypeStruct((2, NP, F), jnp.float32)
                       for _ in range(nch)),
        mesh=_mesh,
        compiler_params=pltpu.CompilerParams(use_tc_tiling_on_sc=False),
        scratch_types=[
            pltpu.VMEM((NEB, EB), jnp.int32),
            pltpu.VMEM((NEB, EB), jnp.int32),
        ] + [pltpu.VMEM((EB, F), jnp.float32)] * NBUF
          + [pltpu.VMEM_SHARED((NP, F), jnp.float32)]
          + [pltpu.SemaphoreType.DMA] * NBUF,
    )
    def mp(*refs):
        h_hbms = refs[:nch]
        srcs_hbm, dsts_hbm, z2_hbm = refs[nch:nch + 3]
        p_hbms = refs[nch + 3:2 * nch + 3]
        idx_s, idx_d = refs[2 * nch + 3:2 * nch + 5]
        rows = refs[2 * nch + 5:2 * nch + 5 + NBUF]
        acc = refs[2 * nch + 5 + NBUF]
        gs = refs[2 * nch + 6 + NBUF:2 * nch + 6 + 2 * NBUF]
        c = lax.axis_index("c")
        s = lax.axis_index("s")
        wid = c * 16 + s
        pltpu.sync_copy(srcs_hbm.at[wid], idx_s)
        pltpu.sync_copy(dsts_hbm.at[wid], idx_d)
        for k in range(nch):
            h_hbm = h_hbms[k]
            pltpu.sync_copy(z2_hbm, acc.at[pl.ds(s * RPT, RPT)])
            for t in range(NBUF - 1):
                pltpu.async_copy(h_hbm.at[idx_s.at[t]], rows[t], gs[t])
            plsc.subcore_barrier()

            def body(j4, carry):
                base = j4 * NBUF
                for t in range(NBUF):
                    jb = base + t
                    pltpu.make_async_copy(
                        h_hbm.at[idx_s.at[jb]], rows[t], gs[t]).wait()
                    pltpu.sync_copy(rows[t], acc.at[idx_d.at[jb]], add=True)
                    tn = (t + NBUF - 1) % NBUF

                    @pl.when(jb + NBUF - 1 < NEB)
                    def _():
                        pltpu.async_copy(
                            h_hbm.at[idx_s.at[jb + NBUF - 1]], rows[tn], gs[tn])

                return carry

            lax.fori_loop(0, NEB // NBUF, body, 0, unroll=False)
            plsc.subcore_barrier()
            pltpu.sync_copy(acc.at[pl.ds(s * RPT, RPT)],
                            p_hbms[k].at[c, pl.ds(s * RPT, RPT)])

    return mp


_mp4 = _make_mp(4)
_mp2 = _make_mp(2)


@functools.partial(
    pl.kernel,
    out_type=jax.ShapeDtypeStruct((2, 128, 128), jnp.float32),
    mesh=_mesh,
    scratch_types=[
        pltpu.VMEM((4, 80), jnp.int32),
        pltpu.VMEM((80, 128), jnp.float32),
        pltpu.VMEM_SHARED((128, 128), jnp.float32),
    ],
)
def _sc_pool(h_hbm, batch_hbm, z2_hbm, pp_hbm, bidx, rows, acc):
    c = lax.axis_index("c")
    s = lax.axis_index("s")
    wid = c * 16 + s
    pltpu.sync_copy(z2_hbm.at[pl.ds(0, PPT)], acc.at[pl.ds(s * PPT, PPT)])
    pltpu.sync_copy(batch_hbm.at[wid], bidx)
    plsc.subcore_barrier()

    def body(jb, carry):
        pltpu.sync_copy(h_hbm.at[pl.ds(wid * 320 + jb * 80, 80)], rows)
        pltpu.sync_copy(rows, acc.at[bidx.at[jb]], add=True)
        return carry

    lax.fori_loop(0, 4, body, 0, unroll=False)
    plsc.subcore_barrier()
    pltpu.sync_copy(acc.at[pl.ds(s * PPT, PPT)],
                    pp_hbm.at[c, pl.ds(s * PPT, PPT)])


def _dinv_blk(deg_ref):
    d = deg_ref[0] + deg_ref[1] + 1.0
    return lax.rsqrt(d)


def _leaky(a):
    return jnp.where(a >= 0, a, NEG * a)


def _k_pre0(x_ref, w_ref, deg_ref, *o_refs):
    dinv = _dinv_blk(deg_ref)
    hs = jnp.dot(x_ref[...], w_ref[...], preferred_element_type=jnp.float32) * dinv
    for k, o in enumerate(o_refs):
        o[...] = hs[:, k * F:(k + 1) * F]


def _k_stats(*refs):
    p_refs = refs[:4]
    h_refs = refs[4:8]
    deg_ref, b_ref, conv_ref, sum_ref, sq_ref = refs[8:]
    i = pl.program_id(0)
    dinv = _dinv_blk(deg_ref)
    parts = [(p[0] + p[1] + h[...]) * dinv for p, h in zip(p_refs, h_refs)]
    conv = jnp.concatenate(parts, axis=1) + b_ref[...]
    conv_ref[...] = conv
    rows = lax.broadcasted_iota(jnp.int32, (R, 1), 0) + i * R
    mask = (rows < N).astype(jnp.float32)
    cm = conv * mask

    @pl.when(i == 0)
    def _():
        sum_ref[...] = jnp.zeros_like(sum_ref)
        sq_ref[...] = jnp.zeros_like(sq_ref)

    sum_ref[...] += jnp.sum(cm, axis=0, keepdims=True)
    sq_ref[...] += jnp.sum(conv * cm, axis=0, keepdims=True)


def _bn_leaky(conv_ref, sum_ref, sq_ref, g_ref, bt_ref):
    m = sum_ref[...] / N
    v = sq_ref[...] / N - m * m
    a = (conv_ref[...] - m) * lax.rsqrt(v + 1e-5) * g_ref[...] + bt_ref[...]
    return _leaky(a)


def _k_pre_mid(conv_ref, sum_ref, sq_ref, g_ref, bt_ref, w_ref, deg_ref,
               *o_refs):
    dinv = _dinv_blk(deg_ref)
    a = _bn_leaky(conv_ref, sum_ref, sq_ref, g_ref, bt_ref)
    hs = jnp.dot(a, w_ref[...], preferred_element_type=jnp.float32) * dinv
    for k, o in enumerate(o_refs):
        o[...] = hs[:, k * F:(k + 1) * F]


def _k_post2(pa_ref, pb_ref, ha_ref, hb_ref, deg_ref, b_ref, o_ref):
    dinv = _dinv_blk(deg_ref)
    ca = (pa_ref[0] + pa_ref[1] + ha_ref[...]) * dinv
    cb = (pb_ref[0] + pb_ref[1] + hb_ref[...]) * dinv
    conv = jnp.concatenate([ca, cb], axis=1) + b_ref[...]
    o_ref[...] = _leaky(conv)


def _k_head(pp_ref, cnt_ref, lw0_ref, lb0_ref, lw1_ref, lb1_ref, o_ref):
    spool = pp_ref[0, :G, :] + pp_ref[1, :G, :]
    cc = cnt_ref[0, :G, :] + cnt_ref[1, :G, :]
    pool = spool / jnp.maximum(cc, 1.0)
    h = jnp.dot(pool, lw0_ref[...], preferred_element_type=jnp.float32) + lb0_ref[...]
    o_ref[...] = jnp.dot(h, lw1_ref[...], preferred_element_type=jnp.float32) + lb1_ref[...]


def _rows_spec(cols):
    return pl.BlockSpec((R, cols), lambda i: (i, 0))


def _full_spec(shape):
    return pl.BlockSpec(shape, lambda i: tuple(0 for _ in shape))


_deg_spec = pl.BlockSpec((2, R, 1), lambda i: (0, i, 0))
_p_spec = pl.BlockSpec((2, R, F), lambda i: (0, i, 0))
_grid = (NP // R,)


def kernel(x, edge_index, batch, cw0, cb0, cw1, cb1, cw2, cb2,
           bg0, bb0, bg1, bb1, lw0, lb0, lw1, lb1):
    f32 = jnp.float32
    x_pad = jnp.concatenate([x, jnp.zeros((NP - N, x.shape[1]), f32)], axis=0)
    srcs = edge_index[0].reshape(NW, NEB, EB)
    dsts = edge_index[1].reshape(NW, NEB, EB)
    batch_r = jnp.concatenate(
        [batch, jnp.full((NP - N,), G, jnp.int32)]).reshape(NW, 4, 80)
    ones128 = jnp.ones((128,), f32)
    z1 = jnp.zeros((RPT,), f32)
    z2 = jnp.zeros((RPT, F), f32)
    z2w = jnp.zeros((RPT, 128), f32)

    deg, cnt = _sc_deg(dsts, batch_r, ones128, z1)
    degv = deg.reshape(2, NP, 1)
    cntv = cnt.reshape(2, 128, 1)

    stats_specs = dict(
        in_specs=[_p_spec] * 4 + [_rows_spec(F)] * 4 + [_deg_spec,
                  _full_spec((1, 256))],
        out_specs=[_rows_spec(256), _full_spec((1, 256)), _full_spec((1, 256))],
        out_shape=[jax.ShapeDtypeStruct((NP, 256), f32),
                   jax.ShapeDtypeStruct((1, 256), f32),
                   jax.ShapeDtypeStruct((1, 256), f32)],
    )

    h0 = pl.pallas_call(
        _k_pre0,
        grid=_grid,
        in_specs=[_rows_spec(128), _full_spec((128, 256)), _deg_spec],
        out_specs=[_rows_spec(F)] * 4,
        out_shape=[jax.ShapeDtypeStruct((NP, F), f32)] * 4,
    )(x_pad, cw0, degv)
    p0 = _mp4(*h0, srcs, dsts, z2)
    conv0, sum0, sq0 = pl.pallas_call(
        _k_stats, grid=_grid, **stats_specs,
    )(*p0, *h0, degv, cb0.reshape(1, 256))

    h1 = pl.pallas_call(
        _k_pre_mid,
        grid=_grid,
        in_specs=[_rows_spec(256), _full_spec((1, 256)), _full_spec((1, 256)),
                  _full_spec((1, 256)), _full_spec((1, 256)),
                  _full_spec((256, 256)), _deg_spec],
        out_specs=[_rows_spec(F)] * 4,
        out_shape=[jax.ShapeDtypeStruct((NP, F), f32)] * 4,
    )(conv0, sum0, sq0, bg0.reshape(1, 256), bb0.reshape(1, 256), cw1, degv)
    p1 = _mp4(*h1, srcs, dsts, z2)
    conv1, sum1, sq1 = pl.pallas_call(
        _k_stats, grid=_grid, **stats_specs,
    )(*p1, *h1, degv, cb1.reshape(1, 256))

    h2s = pl.pallas_call(
        _k_pre_mid,
        grid=_grid,
        in_specs=[_rows_spec(256), _full_spec((1, 256)), _full_spec((1, 256)),
                  _full_spec((1, 256)), _full_spec((1, 256)),
                  _full_spec((256, 128)), _deg_spec],
        out_specs=[_rows_spec(F)] * 2,
        out_shape=[jax.ShapeDtypeStruct((NP, F), f32)] * 2,
    )(conv1, sum1, sq1, bg1.reshape(1, 256), bb1.reshape(1, 256), cw2, degv)
    p2 = _mp2(*h2s, srcs, dsts, z2)
    h2 = pl.pallas_call(
        _k_post2,
        grid=_grid,
        in_specs=[_p_spec, _p_spec, _rows_spec(F), _rows_spec(F), _deg_spec,
                  _full_spec((1, 128))],
        out_specs=_rows_spec(128),
        out_shape=jax.ShapeDtypeStruct((NP, 128), f32),
    )(*p2, *h2s, degv, cb2.reshape(1, 128))

    pp = _sc_pool(h2, batch_r, z2w)
    out = pl.pallas_call(
        _k_head,
        in_specs=[pl.BlockSpec((2, 128, 128), lambda: (0, 0, 0)),
                  pl.BlockSpec((2, 128, 1), lambda: (0, 0, 0)),
                  pl.BlockSpec((128, 64), lambda: (0, 0)),
                  pl.BlockSpec((1, 64), lambda: (0, 0)),
                  pl.BlockSpec((64, 16), lambda: (0, 0)),
                  pl.BlockSpec((1, 16), lambda: (0, 0))],
        out_specs=pl.BlockSpec((G, 16), lambda: (0, 0)),
        out_shape=jax.ShapeDtypeStruct((G, 16), f32),
    )(pp, cntv, lw0, lb0.reshape(1, 64), lw1, lb1.reshape(1, 16))
    return out
